# baseline (speedup 1.0000x reference)
import functools

import jax
import jax.numpy as jnp
from jax import lax
from jax.experimental import pallas as pl
from jax.experimental.pallas import tpu as pltpu
from jax.experimental.pallas import tpu_sc as plsc

B = 16
T = 512
VOCAB = 1024
EMB = 1024
G3 = 3 * EMB


def _proj_body(emb_ref, w_ref, b_ref, out_ref):
    acc = lax.dot_general(
        emb_ref[...], w_ref[...], (((1,), (1,)), ((), ())),
        preferred_element_type=jnp.float32)
    out_ref[...] = acc + b_ref[...]


def _project_table(emb_table, W_ih, b_ih):
    b2 = b_ih.reshape(1, G3)
    return pl.pallas_call(
        _proj_body,
        grid=(3,),
        in_specs=[
            pl.BlockSpec((VOCAB, EMB), lambda j: (0, 0)),
            pl.BlockSpec((EMB, EMB), lambda j: (j, 0)),
            pl.BlockSpec((1, EMB), lambda j: (0, j)),
        ],
        out_specs=pl.BlockSpec((VOCAB, EMB), lambda j: (0, j)),
        out_shape=jax.ShapeDtypeStruct((VOCAB, G3), jnp.float32),
        compiler_params=pltpu.CompilerParams(
            dimension_semantics=("arbitrary",)),
    )(emb_table, W_ih, b2)


_NC = 2
_NS = 16
_NW = _NC * _NS
_ROWS_PER_W = (T * B) // _NW
_CHUNK = 32


def _sc_gather_body(g_hbm, idx_hbm, out_hbm, idx_v, rows_v, sem):
    wid = lax.axis_index("s") * _NC + lax.axis_index("c")
    base = wid * _ROWS_PER_W
    pltpu.sync_copy(idx_hbm.at[pl.ds(base, _ROWS_PER_W)], idx_v)

    def chunk(c, carry):
        off = c * _CHUNK
        pltpu.async_copy(
            g_hbm.at[idx_v.at[pl.ds(off, _CHUNK)]], rows_v, sem).wait()
        pltpu.sync_copy(rows_v, out_hbm.at[pl.ds(base + off, _CHUNK)])
        return carry

    lax.fori_loop(0, _ROWS_PER_W // _CHUNK, chunk, 0)


def _sc_gather(G, idx):
    mesh = plsc.VectorSubcoreMesh(core_axis_name="c", subcore_axis_name="s")
    fn = functools.partial(
        pl.kernel,
        out_type=jax.ShapeDtypeStruct((T * B, G3), jnp.float32),
        mesh=mesh,
        scratch_types=[
            pltpu.VMEM((_ROWS_PER_W,), jnp.int32),
            pltpu.VMEM((_CHUNK, G3), jnp.float32),
            pltpu.SemaphoreType.DMA,
        ],
    )(_sc_gather_body)
    return fn(G, idx)


def _scan_body(tok_ref, gi_ref, whh_ref, bhh_ref, out_ref, alive_ref):
    t = pl.program_id(0)

    @pl.when(t == 0)
    def _init():
        out_ref[...] = jnp.zeros_like(out_ref)
        alive_ref[...] = jnp.ones_like(alive_ref)

    h = out_ref[...]
    gi = gi_ref[...]
    gh = lax.dot_general(
        h.astype(jnp.bfloat16), whh_ref[...], (((1,), (1,)), ((), ())),
        preferred_element_type=jnp.float32) + bhh_ref[...]
    r = jax.nn.sigmoid(gi[:, :EMB] + gh[:, :EMB])
    z = jax.nn.sigmoid(gi[:, EMB:2 * EMB] + gh[:, EMB:2 * EMB])
    n = jnp.tanh(gi[:, 2 * EMB:] + r * gh[:, 2 * EMB:])
    newh = (1.0 - z) * n + z * h

    alive = alive_ref[...]
    tok = tok_ref[...][0]
    out_ref[...] = jnp.where(alive > 0.5, newh, h)
    alive_ref[...] = alive * (tok != 0).astype(jnp.float32)


def _gru_scan(utterance, GI, W_hh, b_hh):
    bhh2 = b_hh.reshape(1, G3)
    return pl.pallas_call(
        _scan_body,
        grid=(T,),
        in_specs=[
            pl.BlockSpec((1, B, 1), lambda t: (t, 0, 0)),
            pl.BlockSpec((B, G3), lambda t: (t, 0)),
            pl.BlockSpec((G3, EMB), lambda t: (0, 0)),
            pl.BlockSpec((1, G3), lambda t: (0, 0)),
        ],
        out_specs=pl.BlockSpec((B, EMB), lambda t: (0, 0)),
        out_shape=jax.ShapeDtypeStruct((B, EMB), jnp.float32),
        scratch_shapes=[pltpu.VMEM((B, 1), jnp.float32)],
        compiler_params=pltpu.CompilerParams(
            dimension_semantics=("arbitrary",)),
    )(utterance.T.reshape(T, B, 1), GI, W_hh.astype(jnp.bfloat16), bhh2)


def kernel(utterance, emb_table, W_ih, W_hh, b_ih, b_hh):
    G = _project_table(emb_table, W_ih, b_ih)
    idx = utterance.T.reshape(T * B)
    GI = _sc_gather(G, idx)
    return _gru_scan(utterance, GI, W_hh, b_hh)


# speedup vs baseline: 1.9751x; 1.9751x over previous
import functools

import jax
import jax.numpy as jnp
from jax import lax
from jax.experimental import pallas as pl
from jax.experimental.pallas import tpu as pltpu
from jax.experimental.pallas import tpu_sc as plsc

B = 16
T = 512
VOCAB = 1024
EMB = 1024
G3 = 3 * EMB
UNROLL = 8
TCHUNK = 4
TSTEPS = T // TCHUNK


def _proj_body(emb_ref, wih_ref, b_ref, whh_ref, out_ref, whht_ref):
    acc = lax.dot_general(
        emb_ref[...].astype(jnp.bfloat16), wih_ref[...].astype(jnp.bfloat16),
        (((1,), (1,)), ((), ())),
        preferred_element_type=jnp.float32)
    out_ref[...] = acc + b_ref[...]
    whht_ref[...] = whh_ref[...].astype(jnp.bfloat16).T


def _project_table(emb_table, W_ih, b_ih, W_hh):
    b2 = b_ih.reshape(1, G3)
    return pl.pallas_call(
        _proj_body,
        grid=(3,),
        in_specs=[
            pl.BlockSpec((VOCAB, EMB), lambda j: (0, 0)),
            pl.BlockSpec((EMB, EMB), lambda j: (j, 0)),
            pl.BlockSpec((1, EMB), lambda j: (0, j)),
            pl.BlockSpec((EMB, EMB), lambda j: (j, 0)),
        ],
        out_specs=[
            pl.BlockSpec((VOCAB, EMB), lambda j: (0, j)),
            pl.BlockSpec((EMB, EMB), lambda j: (0, j)),
        ],
        out_shape=[
            jax.ShapeDtypeStruct((VOCAB, G3), jnp.float32),
            jax.ShapeDtypeStruct((EMB, G3), jnp.bfloat16),
        ],
        compiler_params=pltpu.CompilerParams(
            dimension_semantics=("arbitrary",)),
    )(emb_table, W_ih, b2, W_hh)


_NC = 2
_NS = 16
_NW = _NC * _NS
_CROWS = (T * B) // TCHUNK
_ROWS_PER_W = _CROWS // _NW
_CHUNK = 16
_NCHUNK = _ROWS_PER_W // _CHUNK


def _sc_gather_body(g_hbm, idx_hbm, out_hbm, idx_v, rows0, rows1, gsem0,
                    gsem1, ssem0, ssem1):
    wid = lax.axis_index("s") * _NC + lax.axis_index("c")
    base = wid * _ROWS_PER_W
    pltpu.sync_copy(idx_hbm.at[pl.ds(base, _ROWS_PER_W)], idx_v)

    bufs = (rows0, rows1)
    gsems = (gsem0, gsem1)
    ssems = (ssem0, ssem1)

    def gather(c):
        return pltpu.async_copy(
            g_hbm.at[idx_v.at[pl.ds(c * _CHUNK, _CHUNK)]],
            bufs[c % 2], gsems[c % 2])

    def scatter(c):
        return pltpu.async_copy(
            bufs[c % 2], out_hbm.at[pl.ds(base + c * _CHUNK, _CHUNK)],
            ssems[c % 2])

    pend_g, pend_s = {}, {}
    pend_g[0] = gather(0)
    for c in range(_NCHUNK):
        pend_g[c].wait()
        if c + 1 < _NCHUNK:
            if c >= 1:
                pend_s[c - 1].wait()
            pend_g[c + 1] = gather(c + 1)
        pend_s[c] = scatter(c)
    pend_s[_NCHUNK - 1].wait()
    pend_s[_NCHUNK - 2].wait()


def _sc_gather(G, idx_chunk):
    mesh = plsc.VectorSubcoreMesh(core_axis_name="c", subcore_axis_name="s")
    fn = functools.partial(
        pl.kernel,
        out_type=jax.ShapeDtypeStruct((_CROWS, G3), jnp.float32),
        mesh=mesh,
        scratch_types=[
            pltpu.VMEM((_ROWS_PER_W,), jnp.int32),
            pltpu.VMEM((_CHUNK, G3), jnp.float32),
            pltpu.VMEM((_CHUNK, G3), jnp.float32),
            pltpu.SemaphoreType.DMA,
            pltpu.SemaphoreType.DMA,
            pltpu.SemaphoreType.DMA,
            pltpu.SemaphoreType.DMA,
        ],
    )(_sc_gather_body)
    return fn(G, idx_chunk)


def _scan_body(tok_ref, gi_ref, whh_ref, bhh_ref, h0_ref, al0_ref,
               hout_ref, alout_ref):
    t = pl.program_id(0)

    @pl.when(t == 0)
    def _init():
        hout_ref[...] = h0_ref[...]
        alout_ref[...] = al0_ref[...]

    h = hout_ref[...]
    alive = alout_ref[...]
    toks = tok_ref[...][0]
    gi_all = gi_ref[...]

    for s in range(UNROLL):
        gi = gi_all[s * B:(s + 1) * B]
        tok = toks[s * B:(s + 1) * B]
        gh = lax.dot_general(
            h.astype(jnp.bfloat16), whh_ref[...], (((1,), (0,)), ((), ())),
            preferred_element_type=jnp.float32) + bhh_ref[...]
        r = jax.nn.sigmoid(gi[:, :EMB] + gh[:, :EMB])
        z = jax.nn.sigmoid(gi[:, EMB:2 * EMB] + gh[:, EMB:2 * EMB])
        n = jnp.tanh(gi[:, 2 * EMB:] + r * gh[:, 2 * EMB:])
        newh = (1.0 - z) * n + z * h
        h = jnp.where(alive > 0.5, newh, h)
        alive = alive * (tok != 0).astype(jnp.float32)

    hout_ref[...] = h
    alout_ref[...] = alive


def _gru_scan_chunk(toks3, GI, W_hh_T, bhh2, h0, al0):
    return pl.pallas_call(
        _scan_body,
        grid=(TSTEPS // UNROLL,),
        in_specs=[
            pl.BlockSpec((1, UNROLL * B, 1), lambda t: (t, 0, 0)),
            pl.BlockSpec((UNROLL * B, G3), lambda t: (t, 0)),
            pl.BlockSpec((EMB, G3), lambda t: (0, 0)),
            pl.BlockSpec((1, G3), lambda t: (0, 0)),
            pl.BlockSpec((B, EMB), lambda t: (0, 0)),
            pl.BlockSpec((B, 1), lambda t: (0, 0)),
        ],
        out_specs=[
            pl.BlockSpec((B, EMB), lambda t: (0, 0)),
            pl.BlockSpec((B, 1), lambda t: (0, 0)),
        ],
        out_shape=[
            jax.ShapeDtypeStruct((B, EMB), jnp.float32),
            jax.ShapeDtypeStruct((B, 1), jnp.float32),
        ],
        compiler_params=pltpu.CompilerParams(
            dimension_semantics=("arbitrary",)),
    )(toks3, GI, W_hh_T, bhh2, h0, al0)


def kernel(utterance, emb_table, W_ih, W_hh, b_ih, b_hh):
    G, W_hh_T = _project_table(emb_table, W_ih, b_ih, W_hh)
    idx = utterance.T.reshape(T * B)
    toks3 = utterance.T.reshape(T // UNROLL, UNROLL * B, 1)
    bhh2 = b_hh.reshape(1, G3)
    gsteps = TSTEPS // UNROLL

    h = jnp.zeros((B, EMB), jnp.float32)
    al = jnp.ones((B, 1), jnp.float32)
    for c in range(TCHUNK):
        GI_c = _sc_gather(G, idx[c * _CROWS:(c + 1) * _CROWS])
        h, al = _gru_scan_chunk(
            toks3[c * gsteps:(c + 1) * gsteps], GI_c, W_hh_T, bhh2, h, al)
    return h
